# baseline (speedup 1.0000x reference)
import functools

import jax
import jax.numpy as jnp
from jax.experimental import pallas as pl
from jax.experimental.pallas import tpu as pltpu

TOKENS = 32768
TASK_DIM = 768
HIDDEN_DIM = 128
NUM_EXPERTS = 64
BLOCK = 4096


def _logits_t(x, w1, b1, w2, b2):
    h = jax.lax.dot_general(
        x, w1, (((1,), (1,)), ((), ())),
        preferred_element_type=jnp.float32)
    h = jnp.maximum(h + b1, 0.0)
    logits_t = jax.lax.dot_general(
        w2, h, (((1,), (1,)), ((), ())),
        preferred_element_type=jnp.float32)
    return logits_t + b2


def _top2_softmax(logits_t):

    experts = jax.lax.broadcasted_iota(
        jnp.int32, logits_t.shape, 0).astype(jnp.float32)
    big = jnp.float32(NUM_EXPERTS)
    m1 = jnp.max(logits_t, axis=0, keepdims=True)
    i1 = jnp.min(jnp.where(logits_t == m1, experts, big),
                 axis=0, keepdims=True)
    rest = jnp.where(experts == i1, -jnp.inf, logits_t)
    m2 = jnp.max(rest, axis=0, keepdims=True)
    i2 = jnp.min(jnp.where(rest == m2, experts, big),
                 axis=0, keepdims=True)
    t = jnp.exp(m2 - m1)
    s = 1.0 + t
    wa = 1.0 / s
    wb = t / s
    out_t = jnp.where(experts == i1, wa,
                      jnp.where(experts == i2, wb, 0.0))
    return out_t.T


def _gating_kernel(x_ref, w1_ref, b1_ref, w2_ref, b2_ref, out_ref):
    w1 = w1_ref[...]
    b1 = b1_ref[...]
    w2 = w2_ref[...]
    b2 = b2_ref[...]
    half = BLOCK // 2
    lt_a = _logits_t(x_ref[:half, :], w1, b1, w2, b2)
    lt_b = _logits_t(x_ref[half:, :], w1, b1, w2, b2)
    out_ref[:half, :] = _top2_softmax(lt_a)
    out_ref[half:, :] = _top2_softmax(lt_b)


@functools.partial(jax.jit, static_argnames=("interpret",))
def kernel(task_emb, W1, b1, W2, b2, interpret=False):
    grid = (TOKENS // BLOCK,)
    return pl.pallas_call(
        _gating_kernel,
        grid=grid,
        in_specs=[
            pl.BlockSpec((BLOCK, TASK_DIM), lambda i: (i, 0)),
            pl.BlockSpec((HIDDEN_DIM, TASK_DIM), lambda i: (0, 0)),
            pl.BlockSpec((1, HIDDEN_DIM), lambda i: (0, 0)),
            pl.BlockSpec((NUM_EXPERTS, HIDDEN_DIM), lambda i: (0, 0)),
            pl.BlockSpec((NUM_EXPERTS, 1), lambda i: (0, 0)),
        ],
        out_specs=pl.BlockSpec((BLOCK, NUM_EXPERTS), lambda i: (i, 0)),
        out_shape=jax.ShapeDtypeStruct((TOKENS, NUM_EXPERTS), jnp.float32),
        compiler_params=pltpu.CompilerParams(
            dimension_semantics=("parallel",),
            vmem_limit_bytes=100 * 1024 * 1024),
        interpret=interpret,
    )(task_emb, W1, b1.reshape(1, HIDDEN_DIM), W2,
      b2.reshape(NUM_EXPERTS, 1))


# speedup vs baseline: 1.0744x; 1.0744x over previous
import functools

import jax
import jax.numpy as jnp
from jax.experimental import pallas as pl
from jax.experimental.pallas import tpu as pltpu

TOKENS = 32768
TASK_DIM = 768
HIDDEN_DIM = 128
NUM_EXPERTS = 64
BLOCK = 4096


def _gating_kernel(x_ref, w1_ref, b1_ref, w2_ref, b2_ref, out_ref):
    x = x_ref[...]
    h = jax.lax.dot_general(
        x, w1_ref[...], (((1,), (1,)), ((), ())),
        preferred_element_type=jnp.float32)
    h = jnp.maximum(h + b1_ref[...], 0.0)
    logits_t = jax.lax.dot_general(
        w2_ref[...], h, (((1,), (1,)), ((), ())),
        preferred_element_type=jnp.float32)
    logits_t = logits_t + b2_ref[...]

    experts = jax.lax.broadcasted_iota(
        jnp.int32, logits_t.shape, 0).astype(jnp.float32)
    big = jnp.float32(NUM_EXPERTS)
    m1 = jnp.max(logits_t, axis=0, keepdims=True)
    i1 = jnp.min(jnp.where(logits_t == m1, experts, big),
                 axis=0, keepdims=True)
    rest = jnp.where(experts == i1, -jnp.inf, logits_t)
    m2 = jnp.max(rest, axis=0, keepdims=True)
    i2 = jnp.min(jnp.where(rest == m2, experts, big),
                 axis=0, keepdims=True)
    t = jnp.exp(m2 - m1)
    s = 1.0 + t
    wa = 1.0 / s
    wb = t / s
    out_t = jnp.where(experts == i1, wa,
                      jnp.where(experts == i2, wb, 0.0))
    out_ref[...] = out_t.T


@functools.partial(jax.jit, static_argnames=("interpret",))
def kernel(task_emb, W1, b1, W2, b2, interpret=False):
    grid = (TOKENS // BLOCK,)
    return pl.pallas_call(
        _gating_kernel,
        grid=grid,
        in_specs=[
            pl.BlockSpec((BLOCK, TASK_DIM), lambda i: (i, 0)),
            pl.BlockSpec((HIDDEN_DIM, TASK_DIM), lambda i: (0, 0)),
            pl.BlockSpec((1, HIDDEN_DIM), lambda i: (0, 0)),
            pl.BlockSpec((NUM_EXPERTS, HIDDEN_DIM), lambda i: (0, 0)),
            pl.BlockSpec((NUM_EXPERTS, 1), lambda i: (0, 0)),
        ],
        out_specs=pl.BlockSpec((BLOCK, NUM_EXPERTS), lambda i: (i, 0)),
        out_shape=jax.ShapeDtypeStruct((TOKENS, NUM_EXPERTS), jnp.float32),
        compiler_params=pltpu.CompilerParams(
            dimension_semantics=("parallel",),
            vmem_limit_bytes=100 * 1024 * 1024),
        interpret=interpret,
    )(task_emb, W1, b1.reshape(1, HIDDEN_DIM), W2,
      b2.reshape(NUM_EXPERTS, 1))
